# baseline (speedup 1.0000x reference)
import functools

import jax
import jax.numpy as jnp
from jax import lax
from jax.experimental import pallas as pl
from jax.experimental.pallas import tpu as pltpu
from jax.experimental.pallas import tpu_sc as plsc

N_NODES = 10000
DEG = 32
D = 128
NR = 5

NC = 2
NS = 16
NW = NC * NS
BPW = 320
B_PAD = NW * BPW
RPW = 128
ROUNDS = ((0, 128), (128, 128), (256, 64))
STAGE_ROWS = 624


def _sc_gather(nodes_p, adj4, rat4, feature_table):
  mesh = plsc.VectorSubcoreMesh(core_axis_name="c", subcore_axis_name="s")

  @functools.partial(
      pl.kernel,
      mesh=mesh,
      compiler_params=pltpu.CompilerParams(needs_layout_passes=False),
      out_type=(
          jax.ShapeDtypeStruct((B_PAD, D), jnp.float32),
          jax.ShapeDtypeStruct((B_PAD, D), jnp.float32),
          jax.ShapeDtypeStruct((B_PAD * 8,), jnp.float32),
      ),
      scratch_types=[
          pltpu.VMEM((BPW,), jnp.int32),
          pltpu.VMEM((BPW,), jnp.int32),
          pltpu.VMEM((BPW,), jnp.int32),
          pltpu.VMEM((RPW, D), jnp.int32),
          pltpu.VMEM((DEG, RPW), jnp.int32),
          pltpu.VMEM((RPW, D), jnp.float32),
          pltpu.VMEM((RPW * 8,), jnp.float32),
          pltpu.VMEM_SHARED((N_NODES, D), jnp.float32),
          pltpu.SemaphoreType.DMA,
          pltpu.SemaphoreType.DMA,
          pltpu.SemaphoreType.DMA,
          pltpu.SemaphoreType.DMA,
      ],
  )
  def k(nodes_hbm, adj4_hbm, rat4_hbm, feat_hbm,
        nsum_hbm, self_hbm, aro_hbm,
        idx_v, idxb_v, nmod_v, ar_v, nbt_v, acc_v, cnt_v, feat_s,
        sem_ar, sem_self, sem_acc, sem_stage):
    wid = lax.axis_index("s") * NC + lax.axis_index("c")
    base = wid * BPW
    sid = lax.axis_index("s")

    stage_cp = pltpu.async_copy(
        feat_hbm.at[pl.ds(sid * STAGE_ROWS, STAGE_ROWS)],
        feat_s.at[pl.ds(sid * STAGE_ROWS, STAGE_ROWS)], sem_stage)

    @pl.when(sid == NS - 1)
    def _stage_tail():
      pltpu.sync_copy(
          feat_hbm.at[pl.ds(NS * STAGE_ROWS, N_NODES - NS * STAGE_ROWS)],
          feat_s.at[pl.ds(NS * STAGE_ROWS, N_NODES - NS * STAGE_ROWS)])

    pltpu.sync_copy(nodes_hbm.at[pl.ds(base, BPW)], idx_v)

    zero16 = jnp.zeros((16,), jnp.float32)
    lanes = lax.iota(jnp.int32, 16)

    for g in range(BPW // 16):
      iv = idx_v[pl.ds(g * 16, 16)]
      idxb_v[pl.ds(g * 16, 16)] = iv >> 2
      nmod_v[pl.ds(g * 16, 16)] = (iv & 3) << 5

    ar_cp = pltpu.async_copy(
        adj4_hbm.at[idxb_v.at[pl.ds(0, ROUNDS[0][1])]],
        ar_v.at[pl.ds(0, ROUNDS[0][1])], sem_ar)
    stage_cp.wait()
    plsc.subcore_barrier()

    for r, (o0, rn) in enumerate(ROUNDS):
      ar_cp.wait()

      def tr(j, c):
        for g in range(rn // 16):
          nm = nmod_v[pl.ds(o0 + g * 16, 16)]
          vals = plsc.load_gather(ar_v, [g * 16 + lanes, nm + j])
          nbt_v[j, pl.ds(g * 16, 16)] = vals
        return c

      lax.fori_loop(0, DEG, tr, 0)

      rat_cp = pltpu.async_copy(
          rat4_hbm.at[idxb_v.at[pl.ds(o0, rn)]],
          ar_v.at[pl.ds(0, rn)], sem_ar)

      pltpu.async_copy(
          feat_s.at[idx_v.at[pl.ds(o0, rn)]], acc_v.at[pl.ds(0, rn)],
          sem_self).wait()
      pltpu.sync_copy(acc_v.at[pl.ds(0, rn)],
                      self_hbm.at[pl.ds(base + o0, rn)])

      def zrow(i, c):
        for k8 in range(D // 16):
          acc_v[i, pl.ds(k8 * 16, 16)] = zero16
        return c

      lax.fori_loop(0, rn, zrow, 0)

      def fire(j, c):
        pltpu.async_copy(
            feat_s.at[nbt_v.at[j, pl.ds(0, rn)]],
            acc_v.at[pl.ds(0, rn)], sem_acc, add=True)
        return c

      lax.fori_loop(0, DEG, fire, 0)

      rat_cp.wait()
      for g in range(rn // 16):
        rows_g = g * 16 + lanes
        nm_g = nmod_v[pl.ds(o0 + g * 16, 16)]

        def cbody(j, cs):
          vals = plsc.load_gather(ar_v, [rows_g, nm_g + j])
          return tuple(
              cs[rr] + (vals == rr).astype(jnp.float32) for rr in range(NR))

        counts = lax.fori_loop(
            0, DEG, cbody, tuple(jnp.zeros((16,), jnp.float32)
                                 for _ in range(NR)))
        for rr in range(NR):
          plsc.store_scatter(cnt_v, [rows_g * 8 + rr], counts[rr])

      pltpu.sync_copy(cnt_v.at[pl.ds(0, rn * 8)],
                      aro_hbm.at[pl.ds((base + o0) * 8, rn * 8)])

      if r + 1 < len(ROUNDS):
        o1, rn1 = ROUNDS[r + 1]
        ar_cp = pltpu.async_copy(
            adj4_hbm.at[idxb_v.at[pl.ds(o1, rn1)]],
            ar_v.at[pl.ds(0, rn1)], sem_ar)

      def drain(j, c):
        pltpu.make_async_copy(
            feat_s.at[nbt_v.at[0, pl.ds(0, rn)]],
            acc_v.at[pl.ds(0, rn)], sem_acc).wait()
        return c

      lax.fori_loop(0, DEG, drain, 0)
      pltpu.sync_copy(acc_v.at[pl.ds(0, rn)],
                      nsum_hbm.at[pl.ds(base + o0, rn)])

  return k(nodes_p, adj4, rat4, feature_table)


BB = 2000


def _tc_body(self_ref, nsum_ref, cnt_ref, rtab_ref, wt_ref, b_ref, out_ref):
  rsum = jnp.zeros((BB, D), jnp.float32)
  for r in range(NR):
    rsum = rsum + cnt_ref[:, r:r + 1] * rtab_ref[r:r + 1, :]
  neigh = (nsum_ref[...] + rsum) * (1.0 / DEG)
  out = jnp.dot(self_ref[...], wt_ref[0:D, :],
                preferred_element_type=jnp.float32)
  out += jnp.dot(neigh, wt_ref[D:2 * D, :],
                 preferred_element_type=jnp.float32)
  out_ref[...] = jnp.maximum(out + b_ref[...], 0.0)


def _tc_combine(selfv, nsum, aro, rating_table, Wt, b2):
  return pl.pallas_call(
      _tc_body,
      grid=(N_NODES // BB,),
      in_specs=[
          pl.BlockSpec((BB, D), lambda i: (i, 0)),
          pl.BlockSpec((BB, D), lambda i: (i, 0)),
          pl.BlockSpec((BB, 8), lambda i: (i, 0)),
          pl.BlockSpec((NR, D), lambda i: (0, 0)),
          pl.BlockSpec((2 * D, D), lambda i: (0, 0)),
          pl.BlockSpec((1, D), lambda i: (0, 0)),
      ],
      out_specs=pl.BlockSpec((BB, D), lambda i: (i, 0)),
      out_shape=jax.ShapeDtypeStruct((N_NODES, D), jnp.float32),
  )(selfv, nsum, aro, rating_table, Wt, b2)


def kernel(nodes, adj, ratings, feature_table, rating_table, W, b):
  nodes = nodes.astype(jnp.int32)
  nodes_p = jnp.concatenate(
      [nodes, jnp.zeros((B_PAD - N_NODES,), jnp.int32)])
  adj4 = adj.astype(jnp.int32).reshape(N_NODES // 4, 4 * DEG)
  rat4 = ratings.astype(jnp.int32).reshape(N_NODES // 4, 4 * DEG)
  nsum, selfv, cnt = _sc_gather(nodes_p, adj4, rat4, feature_table)
  return _tc_combine(selfv, nsum, cnt.reshape(B_PAD, 8), rating_table,
                     W.T.astype(jnp.float32), b.reshape(1, D))


# speedup vs baseline: 1.0080x; 1.0080x over previous
import functools

import jax
import jax.numpy as jnp
from jax import lax
from jax.experimental import pallas as pl
from jax.experimental.pallas import tpu as pltpu
from jax.experimental.pallas import tpu_sc as plsc

N_NODES = 10000
DEG = 32
D = 128
NR = 5

NC = 2
NS = 16
NW = NC * NS
BPW = 320
B_PAD = NW * BPW
RPW = 160
ROUNDS = ((0, RPW), (RPW, RPW))
CHUNKS = ((0, 128), (128, 32))
STAGE_ROWS = 624


def _sc_gather(nodes_p, adj4, rat4, feature_table):
  mesh = plsc.VectorSubcoreMesh(core_axis_name="c", subcore_axis_name="s")

  @functools.partial(
      pl.kernel,
      mesh=mesh,
      compiler_params=pltpu.CompilerParams(needs_layout_passes=False),
      out_type=(
          jax.ShapeDtypeStruct((B_PAD, D), jnp.float32),
          jax.ShapeDtypeStruct((B_PAD, D), jnp.float32),
          jax.ShapeDtypeStruct((B_PAD * 8,), jnp.float32),
      ),
      scratch_types=[
          pltpu.VMEM((RPW,), jnp.int32),
          pltpu.VMEM((RPW,), jnp.int32),
          pltpu.VMEM((RPW,), jnp.int32),
          pltpu.VMEM((RPW, D), jnp.int32),
          pltpu.VMEM((DEG, RPW), jnp.int32),
          pltpu.VMEM((RPW, D), jnp.float32),
          pltpu.VMEM((RPW * 4,), jnp.float32),
          pltpu.VMEM_SHARED((N_NODES, D), jnp.float32),
          pltpu.SemaphoreType.DMA,
          pltpu.SemaphoreType.DMA,
          pltpu.SemaphoreType.DMA,
          pltpu.SemaphoreType.DMA,
      ],
  )
  def k(nodes_hbm, adj4_hbm, rat4_hbm, feat_hbm,
        nsum_hbm, self_hbm, aro_hbm,
        idx_v, idxb_v, nmod_v, ar_v, nbt_v, acc_v, cnt_v, feat_s,
        sem_ar, sem_self, sem_acc, sem_stage):
    wid = lax.axis_index("s") * NC + lax.axis_index("c")
    base = wid * BPW
    sid = lax.axis_index("s")

    stage_cp = pltpu.async_copy(
        feat_hbm.at[pl.ds(sid * STAGE_ROWS, STAGE_ROWS)],
        feat_s.at[pl.ds(sid * STAGE_ROWS, STAGE_ROWS)], sem_stage)

    @pl.when(sid == NS - 1)
    def _stage_tail():
      pltpu.sync_copy(
          feat_hbm.at[pl.ds(NS * STAGE_ROWS, N_NODES - NS * STAGE_ROWS)],
          feat_s.at[pl.ds(NS * STAGE_ROWS, N_NODES - NS * STAGE_ROWS)])

    zero16 = jnp.zeros((16,), jnp.float32)
    lanes = lax.iota(jnp.int32, 16)

    def load_round_ids(o0):
      pltpu.sync_copy(nodes_hbm.at[pl.ds(base + o0, RPW)], idx_v)
      for g in range(RPW // 16):
        iv = idx_v[pl.ds(g * 16, 16)]
        idxb_v[pl.ds(g * 16, 16)] = iv >> 2
        nmod_v[pl.ds(g * 16, 16)] = (iv & 3) << 5

    load_round_ids(0)
    ar_cps = [
        pltpu.async_copy(
            adj4_hbm.at[idxb_v.at[pl.ds(o, n)]], ar_v.at[pl.ds(o, n)],
            sem_ar)
        for o, n in CHUNKS
    ]
    stage_cp.wait()
    plsc.subcore_barrier()

    for r, (o0, rn) in enumerate(ROUNDS):
      for cp in ar_cps:
        cp.wait()

      for g in range(RPW // 16):
        rows_t = g * 16 + lanes
        nm_t = nmod_v[pl.ds(g * 16, 16)]

        def tr(j, c):
          vals = plsc.load_gather(ar_v, [rows_t, nm_t + j])
          nbt_v[j, pl.ds(g * 16, 16)] = vals
          return c

        lax.fori_loop(0, DEG, tr, 0)

      rat_cps = [
          pltpu.async_copy(
              rat4_hbm.at[idxb_v.at[pl.ds(o, n)]], ar_v.at[pl.ds(o, n)],
              sem_ar)
          for o, n in CHUNKS
      ]

      for o, n in CHUNKS:
        pltpu.async_copy(
            feat_s.at[idx_v.at[pl.ds(o, n)]], acc_v.at[pl.ds(o, n)],
            sem_self).wait()
      pltpu.sync_copy(acc_v, self_hbm.at[pl.ds(base + o0, RPW)])

      def zrow(i, c):
        for k8 in range(D // 16):
          acc_v[i, pl.ds(k8 * 16, 16)] = zero16
        return c

      lax.fori_loop(0, RPW, zrow, 0)

      def fire(j, c):
        for o, n in CHUNKS:
          pltpu.async_copy(
              feat_s.at[nbt_v.at[j, pl.ds(o, n)]],
              acc_v.at[pl.ds(o, n)], sem_acc, add=True)
        return c

      lax.fori_loop(0, DEG, fire, 0)

      for cp in rat_cps:
        cp.wait()
      for h in range(2):
        for g in range(h * RPW // 32, (h + 1) * RPW // 32):
          rows_g = g * 16 + lanes
          nm_g = nmod_v[pl.ds(g * 16, 16)]

          def cbody(j, cs):
            vals = plsc.load_gather(ar_v, [rows_g, nm_g + j])
            return tuple(
                cs[rr] + (vals == rr).astype(jnp.float32)
                for rr in range(NR))

          counts = lax.fori_loop(
              0, DEG, cbody, tuple(jnp.zeros((16,), jnp.float32)
                                   for _ in range(NR)))
          for rr in range(NR):
            plsc.store_scatter(
                cnt_v, [(rows_g - h * (RPW // 2)) * 8 + rr], counts[rr])

        pltpu.sync_copy(
            cnt_v,
            aro_hbm.at[pl.ds((base + o0) * 8 + h * (RPW * 4), RPW * 4)])

      if r + 1 < len(ROUNDS):
        load_round_ids(ROUNDS[r + 1][0])
        ar_cps = [
            pltpu.async_copy(
                adj4_hbm.at[idxb_v.at[pl.ds(o, n)]], ar_v.at[pl.ds(o, n)],
                sem_ar)
            for o, n in CHUNKS
        ]

      def drain(j, c):
        for o, n in CHUNKS:
          pltpu.make_async_copy(
              feat_s.at[nbt_v.at[0, pl.ds(o, n)]],
              acc_v.at[pl.ds(o, n)], sem_acc).wait()
        return c

      lax.fori_loop(0, DEG, drain, 0)
      pltpu.sync_copy(acc_v, nsum_hbm.at[pl.ds(base + o0, RPW)])

  return k(nodes_p, adj4, rat4, feature_table)


BB = 2000


def _tc_body(self_ref, nsum_ref, cnt_ref, rtab_ref, wt_ref, b_ref, out_ref):
  rsum = jnp.zeros((BB, D), jnp.float32)
  for r in range(NR):
    rsum = rsum + cnt_ref[:, r:r + 1] * rtab_ref[r:r + 1, :]
  neigh = (nsum_ref[...] + rsum) * (1.0 / DEG)
  out = jnp.dot(self_ref[...], wt_ref[0:D, :],
                preferred_element_type=jnp.float32)
  out += jnp.dot(neigh, wt_ref[D:2 * D, :],
                 preferred_element_type=jnp.float32)
  out_ref[...] = jnp.maximum(out + b_ref[...], 0.0)


def _tc_combine(selfv, nsum, aro, rating_table, Wt, b2):
  return pl.pallas_call(
      _tc_body,
      grid=(N_NODES // BB,),
      in_specs=[
          pl.BlockSpec((BB, D), lambda i: (i, 0)),
          pl.BlockSpec((BB, D), lambda i: (i, 0)),
          pl.BlockSpec((BB, 8), lambda i: (i, 0)),
          pl.BlockSpec((NR, D), lambda i: (0, 0)),
          pl.BlockSpec((2 * D, D), lambda i: (0, 0)),
          pl.BlockSpec((1, D), lambda i: (0, 0)),
      ],
      out_specs=pl.BlockSpec((BB, D), lambda i: (i, 0)),
      out_shape=jax.ShapeDtypeStruct((N_NODES, D), jnp.float32),
  )(selfv, nsum, aro, rating_table, Wt, b2)


def kernel(nodes, adj, ratings, feature_table, rating_table, W, b):
  nodes = nodes.astype(jnp.int32)
  nodes_p = jnp.concatenate(
      [nodes, jnp.zeros((B_PAD - N_NODES,), jnp.int32)])
  adj4 = adj.astype(jnp.int32).reshape(N_NODES // 4, 4 * DEG)
  rat4 = ratings.astype(jnp.int32).reshape(N_NODES // 4, 4 * DEG)
  nsum, selfv, cnt = _sc_gather(nodes_p, adj4, rat4, feature_table)
  return _tc_combine(selfv, nsum, cnt.reshape(B_PAD, 8), rating_table,
                     W.T.astype(jnp.float32), b.reshape(1, D))
